# baseline (speedup 1.0000x reference)
import functools

import jax
import jax.numpy as jnp
from jax import lax
from jax.experimental import pallas as pl
from jax.experimental.pallas import tpu as pltpu
from jax.experimental.pallas import tpu_sc as plsc

N = 10000
E = 320000
D = 128
H = 128
C = 64

NC = 2
NS = 16
NW = NC * NS
NPAD = 10240
EPW = 10240
CHUNKS = EPW // 128
EPAD = EPW * NW
ROWS_PER_TILE = NPAD // NS
BM = 1024


def _sc_mesh():
    return plsc.VectorSubcoreMesh(core_axis_name="c", subcore_axis_name="s")


def _build_segsum(width):
    @functools.partial(
        pl.kernel,
        out_type=jax.ShapeDtypeStruct((NC, NPAD, width), jnp.float32),
        mesh=_sc_mesh(),
        scratch_types=[
            pltpu.VMEM((CHUNKS, 128), jnp.int32),
            pltpu.VMEM((CHUNKS, 128), jnp.int32),
            pltpu.VMEM((128, width), jnp.float32),
            pltpu.VMEM_SHARED((NPAD, width), jnp.float32),
            pltpu.SemaphoreType.DMA,
        ],
        compiler_params=pltpu.CompilerParams(use_tc_tiling_on_sc=False),
    )
    def k(hs_hbm, row_hbm, col_hbm, zeros_hbm, out_hbm, idxr, idxc, rows,
          acc, sem):
        cid = lax.axis_index("c")
        sid = lax.axis_index("s")
        wid = sid * NC + cid
        base = sid * ROWS_PER_TILE
        pltpu.sync_copy(zeros_hbm, rows)
        for kk in range(ROWS_PER_TILE // 128):
            pltpu.sync_copy(rows, acc.at[pl.ds(base + kk * 128, 128)])
        pltpu.sync_copy(row_hbm.at[wid], idxr)
        pltpu.sync_copy(col_hbm.at[wid], idxc)
        plsc.subcore_barrier()

        def step(j, carry):
            pltpu.async_copy(hs_hbm.at[idxr.at[j]], rows, sem).wait()
            pltpu.sync_copy(rows, acc.at[idxc.at[j]], add=True)
            return carry

        lax.fori_loop(0, CHUNKS, step, 0)
        plsc.subcore_barrier()
        for kk in range(ROWS_PER_TILE // 128):
            sl = pl.ds(base + kk * 128, 128)
            pltpu.sync_copy(acc.at[sl], rows)
            pltpu.sync_copy(rows, out_hbm.at[cid, sl])

    return k


def _build_degree():
    @functools.partial(
        pl.kernel,
        out_type=jax.ShapeDtypeStruct((NC, NPAD, 16), jnp.float32),
        mesh=_sc_mesh(),
        scratch_types=[
            pltpu.VMEM((CHUNKS, 128), jnp.int32),
            pltpu.VMEM((128, 16), jnp.float32),
            pltpu.VMEM_SHARED((NPAD, 16), jnp.float32),
        ],
        compiler_params=pltpu.CompilerParams(use_tc_tiling_on_sc=False),
    )
    def k(col_hbm, zeros_hbm, ones_hbm, out_hbm, idxc, buf, acc):
        cid = lax.axis_index("c")
        sid = lax.axis_index("s")
        wid = sid * NC + cid
        base = sid * ROWS_PER_TILE
        pltpu.sync_copy(zeros_hbm, buf)
        for kk in range(ROWS_PER_TILE // 128):
            pltpu.sync_copy(buf, acc.at[pl.ds(base + kk * 128, 128)])
        pltpu.sync_copy(col_hbm.at[wid], idxc)
        pltpu.sync_copy(ones_hbm, buf)
        plsc.subcore_barrier()

        def step(j, carry):
            pltpu.sync_copy(buf, acc.at[idxc.at[j]], add=True)
            return carry

        lax.fori_loop(0, CHUNKS, step, 0)
        plsc.subcore_barrier()
        for kk in range(ROWS_PER_TILE // 128):
            sl = pl.ds(base + kk * 128, 128)
            pltpu.sync_copy(acc.at[sl], buf)
            pltpu.sync_copy(buf, out_hbm.at[cid, sl])

    return k


def _dinv_body(d0, d1, o):
    o[...] = lax.rsqrt(d0[...] + d1[...] + 1.0)


_dinv_call = pl.pallas_call(
    _dinv_body,
    out_shape=jax.ShapeDtypeStruct((NPAD // 128, 128), jnp.float32),
)


def _mm_body(x, w, dinv, o):
    o[...] = jnp.dot(x[...], w[...],
                     preferred_element_type=jnp.float32) * dinv[...]


_mm_call = pl.pallas_call(
    _mm_body,
    grid=(NPAD // BM,),
    in_specs=[
        pl.BlockSpec((BM, D), lambda i: (i, 0)),
        pl.BlockSpec((D, H), lambda i: (0, 0)),
        pl.BlockSpec((BM, 1), lambda i: (i, 0)),
    ],
    out_specs=pl.BlockSpec((BM, H), lambda i: (i, 0)),
    out_shape=jax.ShapeDtypeStruct((NPAD, H), jnp.float32),
)


def _build_combine_mm(has_res, hin, hout):
    def body(t0, t1, hs, dinv, b, *rest):
        if has_res:
            res, w, xo, ho = rest
        else:
            w, xo, ho = rest
        u = dinv[...] * (t0[...] + t1[...] + hs[...]) + b[...]
        if has_res:
            u = u + res[...]
        xn = jnp.where(u > 0, u, jnp.exp(jnp.minimum(u, 0.0)) - 1.0)
        xo[...] = xn
        ho[...] = jnp.dot(xn, w[...],
                          preferred_element_type=jnp.float32) * dinv[...]

    blk = pl.BlockSpec((BM, hin), lambda i: (i, 0))
    in_specs = [blk, blk, blk,
                pl.BlockSpec((BM, 1), lambda i: (i, 0)),
                pl.BlockSpec((1, hin), lambda i: (0, 0))]
    if has_res:
        in_specs.append(blk)
    in_specs.append(pl.BlockSpec((hin, hout), lambda i: (0, 0)))
    return pl.pallas_call(
        body,
        grid=(NPAD // BM,),
        in_specs=in_specs,
        out_specs=[blk, pl.BlockSpec((BM, hout), lambda i: (i, 0))],
        out_shape=[jax.ShapeDtypeStruct((NPAD, hin), jnp.float32),
                   jax.ShapeDtypeStruct((NPAD, hout), jnp.float32)],
    )


def _final_body(t0, t1, hs, dinv, b, o):
    o[...] = dinv[...] * (t0[...] + t1[...] + hs[...]) + b[...]


_final_call = pl.pallas_call(
    _final_body,
    grid=(NPAD // BM,),
    in_specs=[
        pl.BlockSpec((BM, C), lambda i: (i, 0)),
        pl.BlockSpec((BM, C), lambda i: (i, 0)),
        pl.BlockSpec((BM, C), lambda i: (i, 0)),
        pl.BlockSpec((BM, 1), lambda i: (i, 0)),
        pl.BlockSpec((1, C), lambda i: (0, 0)),
    ],
    out_specs=pl.BlockSpec((BM, C), lambda i: (i, 0)),
    out_shape=jax.ShapeDtypeStruct((NPAD, C), jnp.float32),
)


def kernel(x, edge_index, batch, W1, b1, W2, b2, W3, b3):
    f32 = jnp.float32
    row = edge_index[0].astype(jnp.int32)
    col = edge_index[1].astype(jnp.int32)
    pad = jnp.full((EPAD - E,), N, jnp.int32)
    rowp = jnp.concatenate([row, pad]).reshape(NW, CHUNKS, 128)
    colp = jnp.concatenate([col, pad]).reshape(NW, CHUNKS, 128)
    zeros128 = jnp.zeros((128, 128), f32)
    zeros64 = jnp.zeros((128, 64), f32)
    zeros16 = jnp.zeros((128, 16), f32)
    ones16 = jnp.ones((128, 16), f32)
    x_pad = jnp.pad(x, ((0, NPAD - N), (0, 0)))

    degp = _build_degree()(colp, zeros16, ones16)
    d0 = degp[0, :, 0].reshape(NPAD // 128, 128)
    d1 = degp[1, :, 0].reshape(NPAD // 128, 128)
    dinv = _dinv_call(d0, d1).reshape(NPAD, 1)

    hs1 = _mm_call(x_pad, W1, dinv)
    t1 = _build_segsum(H)(hs1, rowp, colp, zeros128)
    x1, hs2 = _build_combine_mm(False, H, H)(
        t1[0], t1[1], hs1, dinv, b1.reshape(1, H), W2)
    t2 = _build_segsum(H)(hs2, rowp, colp, zeros128)
    x2, hs3 = _build_combine_mm(True, H, C)(
        t2[0], t2[1], hs2, dinv, b2.reshape(1, H), x1, W3)
    t3 = _build_segsum(C)(hs3, rowp, colp, zeros64)
    out = _final_call(t3[0], t3[1], hs3, dinv, b3.reshape(1, C))
    return out[:N]


# speedup vs baseline: 2.3979x; 2.3979x over previous
import functools

import jax
import jax.numpy as jnp
from jax import lax
from jax.experimental import pallas as pl
from jax.experimental.pallas import tpu as pltpu
from jax.experimental.pallas import tpu_sc as plsc

N = 10000
E = 320000
D = 128
H = 128
C = 64

NC = 2
NS = 16
NW = NC * NS
NPAD = 10240
EPW = 10240
CHUNKS = EPW // 128
EPAD = EPW * NW
ROWS_PER_TILE = NPAD // NS
BM = 1024


def _sc_mesh():
    return plsc.VectorSubcoreMesh(core_axis_name="c", subcore_axis_name="s")


def _build_segsum(width):
    @functools.partial(
        pl.kernel,
        out_type=jax.ShapeDtypeStruct((NC, NPAD, width), jnp.float32),
        mesh=_sc_mesh(),
        scratch_types=[
            pltpu.VMEM((CHUNKS, 128), jnp.int32),
            pltpu.VMEM((CHUNKS, 128), jnp.int32),
            pltpu.VMEM((128, width), jnp.float32),
            pltpu.VMEM_SHARED((NPAD, width), jnp.float32),
            pltpu.SemaphoreType.DMA,
        ],
        compiler_params=pltpu.CompilerParams(use_tc_tiling_on_sc=False),
    )
    def k(hs_hbm, row_hbm, col_hbm, zeros_hbm, out_hbm, idxr, idxc, rows,
          acc, sem):
        cid = lax.axis_index("c")
        sid = lax.axis_index("s")
        wid = sid * NC + cid
        base = sid * ROWS_PER_TILE
        pltpu.sync_copy(zeros_hbm, rows)
        for kk in range(ROWS_PER_TILE // 128):
            pltpu.sync_copy(rows, acc.at[pl.ds(base + kk * 128, 128)])
        pltpu.sync_copy(row_hbm.at[wid], idxr)
        pltpu.sync_copy(col_hbm.at[wid], idxc)
        plsc.subcore_barrier()

        def step(j, carry):
            pltpu.async_copy(hs_hbm.at[idxr.at[j]], rows, sem).wait()
            pltpu.sync_copy(rows, acc.at[idxc.at[j]], add=True)
            return carry

        lax.fori_loop(0, CHUNKS, step, 0)
        plsc.subcore_barrier()
        for kk in range(ROWS_PER_TILE // 128):
            sl = pl.ds(base + kk * 128, 128)
            pltpu.sync_copy(acc.at[sl], rows)
            pltpu.sync_copy(rows, out_hbm.at[cid, sl])

    return k


def _build_degree():
    @functools.partial(
        pl.kernel,
        out_type=jax.ShapeDtypeStruct((NC, NPAD, 16), jnp.float32),
        mesh=_sc_mesh(),
        scratch_types=[
            pltpu.VMEM((CHUNKS, 128), jnp.int32),
            pltpu.VMEM((128, 16), jnp.float32),
            pltpu.VMEM_SHARED((NPAD, 16), jnp.float32),
        ],
        compiler_params=pltpu.CompilerParams(use_tc_tiling_on_sc=False),
    )
    def k(col_hbm, zeros_hbm, ones_hbm, out_hbm, idxc, buf, acc):
        cid = lax.axis_index("c")
        sid = lax.axis_index("s")
        wid = sid * NC + cid
        base = sid * ROWS_PER_TILE
        pltpu.sync_copy(zeros_hbm, buf)
        for kk in range(ROWS_PER_TILE // 128):
            pltpu.sync_copy(buf, acc.at[pl.ds(base + kk * 128, 128)])
        pltpu.sync_copy(col_hbm.at[wid], idxc)
        pltpu.sync_copy(ones_hbm, buf)
        plsc.subcore_barrier()

        def step(j, carry):
            pltpu.sync_copy(buf, acc.at[idxc.at[j]], add=True)
            return carry

        lax.fori_loop(0, CHUNKS, step, 0)
        plsc.subcore_barrier()
        for kk in range(ROWS_PER_TILE // 128):
            sl = pl.ds(base + kk * 128, 128)
            pltpu.sync_copy(acc.at[sl], buf)
            pltpu.sync_copy(buf, out_hbm.at[cid, sl])

    return k


def _dinv_body(d0, d1, o):
    o[...] = lax.rsqrt(d0[...] + d1[...] + 1.0)


_dinv_call = pl.pallas_call(
    _dinv_body,
    out_shape=jax.ShapeDtypeStruct((NPAD // 128, 128), jnp.float32),
)


def _mm_body(x, w, dinv, o):
    o[...] = jnp.dot(x[...], w[...],
                     preferred_element_type=jnp.float32) * dinv[...]


_mm_call = pl.pallas_call(
    _mm_body,
    grid=(NPAD // BM,),
    in_specs=[
        pl.BlockSpec((BM, D), lambda i: (i, 0)),
        pl.BlockSpec((D, H), lambda i: (0, 0)),
        pl.BlockSpec((BM, 1), lambda i: (i, 0)),
    ],
    out_specs=pl.BlockSpec((BM, H), lambda i: (i, 0)),
    out_shape=jax.ShapeDtypeStruct((NPAD, H), jnp.float32),
)


def _build_combine_mm(has_res, hin, hout):
    def body(t0, t1, hs, dinv, b, *rest):
        if has_res:
            res, w, xo, ho = rest
        else:
            w, xo, ho = rest
        u = dinv[...] * (t0[...] + t1[...] + hs[...]) + b[...]
        if has_res:
            u = u + res[...]
        xn = jnp.where(u > 0, u, jnp.exp(jnp.minimum(u, 0.0)) - 1.0)
        xo[...] = xn
        ho[...] = jnp.dot(xn, w[...],
                          preferred_element_type=jnp.float32) * dinv[...]

    blk = pl.BlockSpec((BM, hin), lambda i: (i, 0))
    in_specs = [blk, blk, blk,
                pl.BlockSpec((BM, 1), lambda i: (i, 0)),
                pl.BlockSpec((1, hin), lambda i: (0, 0))]
    if has_res:
        in_specs.append(blk)
    in_specs.append(pl.BlockSpec((hin, hout), lambda i: (0, 0)))
    return pl.pallas_call(
        body,
        grid=(NPAD // BM,),
        in_specs=in_specs,
        out_specs=[blk, pl.BlockSpec((BM, hout), lambda i: (i, 0))],
        out_shape=[jax.ShapeDtypeStruct((NPAD, hin), jnp.float32),
                   jax.ShapeDtypeStruct((NPAD, hout), jnp.float32)],
    )


def _final_body(t0, t1, hs, dinv, b, o):
    o[...] = dinv[...] * (t0[...] + t1[...] + hs[...]) + b[...]


_final_call = pl.pallas_call(
    _final_body,
    grid=(NPAD // BM,),
    in_specs=[
        pl.BlockSpec((BM, C), lambda i: (i, 0)),
        pl.BlockSpec((BM, C), lambda i: (i, 0)),
        pl.BlockSpec((BM, C), lambda i: (i, 0)),
        pl.BlockSpec((BM, 1), lambda i: (i, 0)),
        pl.BlockSpec((1, C), lambda i: (0, 0)),
    ],
    out_specs=pl.BlockSpec((BM, C), lambda i: (i, 0)),
    out_shape=jax.ShapeDtypeStruct((NPAD, C), jnp.float32),
)


def kernel(x, edge_index, batch, W1, b1, W2, b2, W3, b3):
    f32 = jnp.float32
    row = edge_index[0].astype(jnp.int32)
    col = edge_index[1].astype(jnp.int32)
    pad = N + (jnp.arange(EPAD - E, dtype=jnp.int32) % (NPAD - N))
    rowp = jnp.concatenate([row, pad]).reshape(NW, CHUNKS, 128)
    colp = jnp.concatenate([col, pad]).reshape(NW, CHUNKS, 128)
    zeros128 = jnp.zeros((128, 128), f32)
    zeros64 = jnp.zeros((128, 64), f32)
    zeros16 = jnp.zeros((128, 16), f32)
    ones16 = jnp.ones((128, 16), f32)
    x_pad = jnp.pad(x, ((0, NPAD - N), (0, 0)))

    degp = _build_degree()(colp, zeros16, ones16)
    d0 = degp[0, :, 0].reshape(NPAD // 128, 128)
    d1 = degp[1, :, 0].reshape(NPAD // 128, 128)
    dinv = _dinv_call(d0, d1).reshape(NPAD, 1)

    hs1 = _mm_call(x_pad, W1, dinv)
    t1 = _build_segsum(H)(hs1, rowp, colp, zeros128)
    x1, hs2 = _build_combine_mm(False, H, H)(
        t1[0], t1[1], hs1, dinv, b1.reshape(1, H), W2)
    t2 = _build_segsum(H)(hs2, rowp, colp, zeros128)
    x2, hs3 = _build_combine_mm(True, H, C)(
        t2[0], t2[1], hs2, dinv, b2.reshape(1, H), x1, W3)
    t3 = _build_segsum(C)(hs3, rowp, colp, zeros64)
    out = _final_call(t3[0], t3[1], hs3, dinv, b3.reshape(1, C))
    return out[:N]


# speedup vs baseline: 3.3496x; 1.3968x over previous
import functools

import jax
import jax.numpy as jnp
from jax import lax
from jax.experimental import pallas as pl
from jax.experimental.pallas import tpu as pltpu
from jax.experimental.pallas import tpu_sc as plsc

N = 10000
E = 320000
D = 128
H = 128
C = 64

NC = 2
NS = 16
NW = NC * NS
NPAD = 10240
EPW = 10240
CHUNKS = EPW // 128
EPAD = EPW * NW
ROWS_PER_TILE = NPAD // NS
BM = 1024


def _sc_mesh():
    return plsc.VectorSubcoreMesh(core_axis_name="c", subcore_axis_name="s")


def _build_segsum(width):
    @functools.partial(
        pl.kernel,
        out_type=jax.ShapeDtypeStruct((NC, NPAD, width), jnp.float32),
        mesh=_sc_mesh(),
        scratch_types=[
            pltpu.VMEM((CHUNKS // 2, 128), jnp.int32),
            pltpu.VMEM((CHUNKS // 2, 128), jnp.int32),
            pltpu.VMEM((128, width), jnp.float32),
            pltpu.VMEM((128, width), jnp.float32),
            pltpu.VMEM_SHARED((NPAD, width), jnp.float32),
            pltpu.SemaphoreType.DMA,
            pltpu.SemaphoreType.DMA,
        ],
        compiler_params=pltpu.CompilerParams(use_tc_tiling_on_sc=False),
    )
    def k(hs_hbm, row_hbm, col_hbm, zeros_hbm, out_hbm, idxr, idxc, rows0,
          rows1, acc, sem0, sem1):
        cid = lax.axis_index("c")
        sid = lax.axis_index("s")
        wid = sid * NC + cid
        base = sid * ROWS_PER_TILE
        cpp = CHUNKS // 2
        pltpu.sync_copy(zeros_hbm, rows0)
        for kk in range(ROWS_PER_TILE // 128):
            pltpu.sync_copy(rows0, acc.at[pl.ds(base + kk * 128, 128)])
        plsc.subcore_barrier()

        for p in range(2):
            pltpu.sync_copy(row_hbm.at[wid, pl.ds(p * cpp, cpp)], idxr)
            pltpu.sync_copy(col_hbm.at[wid, pl.ds(p * cpp, cpp)], idxc)
            pltpu.async_copy(hs_hbm.at[idxr.at[0]], rows0, sem0)
            pltpu.async_copy(hs_hbm.at[idxr.at[1]], rows1, sem1)

            def step(k2, carry):
                j = 2 * k2
                pltpu.make_async_copy(hs_hbm.at[idxr.at[j]], rows0,
                                      sem0).wait()
                pltpu.sync_copy(rows0, acc.at[idxc.at[j]], add=True)

                @pl.when(j + 2 < cpp)
                def _():
                    pltpu.async_copy(hs_hbm.at[idxr.at[j + 2]], rows0, sem0)

                pltpu.make_async_copy(hs_hbm.at[idxr.at[j + 1]], rows1,
                                      sem1).wait()
                pltpu.sync_copy(rows1, acc.at[idxc.at[j + 1]], add=True)

                @pl.when(j + 3 < cpp)
                def _():
                    pltpu.async_copy(hs_hbm.at[idxr.at[j + 3]], rows1, sem1)

                return carry

            lax.fori_loop(0, cpp // 2, step, 0)
        plsc.subcore_barrier()
        for kk in range(ROWS_PER_TILE // 128):
            sl = pl.ds(base + kk * 128, 128)
            pltpu.sync_copy(acc.at[sl], rows0)
            pltpu.sync_copy(rows0, out_hbm.at[cid, sl])

    return k


def _build_degree():
    @functools.partial(
        pl.kernel,
        out_type=jax.ShapeDtypeStruct((NC, NPAD, 16), jnp.float32),
        mesh=_sc_mesh(),
        scratch_types=[
            pltpu.VMEM((CHUNKS, 128), jnp.int32),
            pltpu.VMEM((128, 16), jnp.float32),
            pltpu.VMEM_SHARED((NPAD, 16), jnp.float32),
        ],
        compiler_params=pltpu.CompilerParams(use_tc_tiling_on_sc=False),
    )
    def k(col_hbm, zeros_hbm, ones_hbm, out_hbm, idxc, buf, acc):
        cid = lax.axis_index("c")
        sid = lax.axis_index("s")
        wid = sid * NC + cid
        base = sid * ROWS_PER_TILE
        pltpu.sync_copy(zeros_hbm, buf)
        for kk in range(ROWS_PER_TILE // 128):
            pltpu.sync_copy(buf, acc.at[pl.ds(base + kk * 128, 128)])
        pltpu.sync_copy(col_hbm.at[wid], idxc)
        pltpu.sync_copy(ones_hbm, buf)
        plsc.subcore_barrier()

        def step(j, carry):
            pltpu.sync_copy(buf, acc.at[idxc.at[j]], add=True)
            return carry

        lax.fori_loop(0, CHUNKS, step, 0)
        plsc.subcore_barrier()
        for kk in range(ROWS_PER_TILE // 128):
            sl = pl.ds(base + kk * 128, 128)
            pltpu.sync_copy(acc.at[sl], buf)
            pltpu.sync_copy(buf, out_hbm.at[cid, sl])

    return k


def _dinv_body(d0, d1, o):
    o[...] = lax.rsqrt(d0[...] + d1[...] + 1.0)


_dinv_call = pl.pallas_call(
    _dinv_body,
    out_shape=jax.ShapeDtypeStruct((NPAD // 128, 128), jnp.float32),
)


def _mm_body(x, w, dinv, o):
    o[...] = jnp.dot(x[...], w[...],
                     preferred_element_type=jnp.float32) * dinv[...]


_mm_call = pl.pallas_call(
    _mm_body,
    grid=(NPAD // BM,),
    in_specs=[
        pl.BlockSpec((BM, D), lambda i: (i, 0)),
        pl.BlockSpec((D, H), lambda i: (0, 0)),
        pl.BlockSpec((BM, 1), lambda i: (i, 0)),
    ],
    out_specs=pl.BlockSpec((BM, H), lambda i: (i, 0)),
    out_shape=jax.ShapeDtypeStruct((NPAD, H), jnp.float32),
)


def _build_combine_mm(has_res, hin, hout):
    def body(t0, t1, hs, dinv, b, *rest):
        if has_res:
            res, w, xo, ho = rest
        else:
            w, xo, ho = rest
        u = dinv[...] * (t0[...] + t1[...] + hs[...]) + b[...]
        if has_res:
            u = u + res[...]
        xn = jnp.where(u > 0, u, jnp.exp(jnp.minimum(u, 0.0)) - 1.0)
        xo[...] = xn
        ho[...] = jnp.dot(xn, w[...],
                          preferred_element_type=jnp.float32) * dinv[...]

    blk = pl.BlockSpec((BM, hin), lambda i: (i, 0))
    in_specs = [blk, blk, blk,
                pl.BlockSpec((BM, 1), lambda i: (i, 0)),
                pl.BlockSpec((1, hin), lambda i: (0, 0))]
    if has_res:
        in_specs.append(blk)
    in_specs.append(pl.BlockSpec((hin, hout), lambda i: (0, 0)))
    return pl.pallas_call(
        body,
        grid=(NPAD // BM,),
        in_specs=in_specs,
        out_specs=[blk, pl.BlockSpec((BM, hout), lambda i: (i, 0))],
        out_shape=[jax.ShapeDtypeStruct((NPAD, hin), jnp.float32),
                   jax.ShapeDtypeStruct((NPAD, hout), jnp.float32)],
    )


def _final_body(t0, t1, hs, dinv, b, o):
    o[...] = dinv[...] * (t0[...] + t1[...] + hs[...]) + b[...]


_final_call = pl.pallas_call(
    _final_body,
    grid=(NPAD // BM,),
    in_specs=[
        pl.BlockSpec((BM, C), lambda i: (i, 0)),
        pl.BlockSpec((BM, C), lambda i: (i, 0)),
        pl.BlockSpec((BM, C), lambda i: (i, 0)),
        pl.BlockSpec((BM, 1), lambda i: (i, 0)),
        pl.BlockSpec((1, C), lambda i: (0, 0)),
    ],
    out_specs=pl.BlockSpec((BM, C), lambda i: (i, 0)),
    out_shape=jax.ShapeDtypeStruct((NPAD, C), jnp.float32),
)


def kernel(x, edge_index, batch, W1, b1, W2, b2, W3, b3):
    f32 = jnp.float32
    row = edge_index[0].astype(jnp.int32)
    col = edge_index[1].astype(jnp.int32)
    pad = N + (jnp.arange(EPAD - E, dtype=jnp.int32) % (NPAD - N))
    rowp = jnp.concatenate([row, pad]).reshape(NW, CHUNKS, 128)
    colp = jnp.concatenate([col, pad]).reshape(NW, CHUNKS, 128)
    zeros128 = jnp.zeros((128, 128), f32)
    zeros64 = jnp.zeros((128, 64), f32)
    zeros16 = jnp.zeros((128, 16), f32)
    ones16 = jnp.ones((128, 16), f32)
    x_pad = jnp.pad(x, ((0, NPAD - N), (0, 0)))

    degp = _build_degree()(colp, zeros16, ones16)
    d0 = degp[0, :, 0].reshape(NPAD // 128, 128)
    d1 = degp[1, :, 0].reshape(NPAD // 128, 128)
    dinv = _dinv_call(d0, d1).reshape(NPAD, 1)

    hs1 = _mm_call(x_pad, W1, dinv)
    t1 = _build_segsum(H)(hs1, rowp, colp, zeros128)
    x1, hs2 = _build_combine_mm(False, H, H)(
        t1[0], t1[1], hs1, dinv, b1.reshape(1, H), W2)
    t2 = _build_segsum(H)(hs2, rowp, colp, zeros128)
    x2, hs3 = _build_combine_mm(True, H, C)(
        t2[0], t2[1], hs2, dinv, b2.reshape(1, H), x1, W3)
    t3 = _build_segsum(C)(hs3, rowp, colp, zeros64)
    out = _final_call(t3[0], t3[1], hs3, dinv, b3.reshape(1, C))
    return out[:N]


# speedup vs baseline: 3.3539x; 1.0013x over previous
import functools

import jax
import jax.numpy as jnp
from jax import lax
from jax.experimental import pallas as pl
from jax.experimental.pallas import tpu as pltpu
from jax.experimental.pallas import tpu_sc as plsc

N = 10000
E = 320000
D = 128
H = 128
C = 64

NC = 2
NS = 16
NW = NC * NS
NPAD = 10240
EPW = 10240
CHUNKS = EPW // 128
EPAD = EPW * NW
ROWS_PER_TILE = NPAD // NS
BM = 1024


def _sc_mesh():
    return plsc.VectorSubcoreMesh(core_axis_name="c", subcore_axis_name="s")


def _build_segsum(width):
    @functools.partial(
        pl.kernel,
        out_type=jax.ShapeDtypeStruct((NC, NPAD, width), jnp.float32),
        mesh=_sc_mesh(),
        scratch_types=[
            pltpu.VMEM((CHUNKS // 2, 128), jnp.int32),
            pltpu.VMEM((CHUNKS // 2, 128), jnp.int32),
            pltpu.VMEM((128, width), jnp.float32),
            pltpu.VMEM((128, width), jnp.float32),
            pltpu.VMEM_SHARED((NPAD, width), jnp.float32),
            pltpu.SemaphoreType.DMA,
            pltpu.SemaphoreType.DMA,
        ],
        compiler_params=pltpu.CompilerParams(use_tc_tiling_on_sc=False),
    )
    def k(hs_hbm, row_hbm, col_hbm, zeros_hbm, out_hbm, idxr, idxc, rows0,
          rows1, acc, sem0, sem1):
        cid = lax.axis_index("c")
        sid = lax.axis_index("s")
        wid = sid * NC + cid
        base = sid * ROWS_PER_TILE
        cpp = CHUNKS // 2
        pltpu.sync_copy(zeros_hbm, rows0)
        for kk in range(ROWS_PER_TILE // 128):
            pltpu.sync_copy(rows0, acc.at[pl.ds(base + kk * 128, 128)])
        plsc.subcore_barrier()

        for p in range(2):
            pltpu.sync_copy(row_hbm.at[wid, pl.ds(p * cpp, cpp)], idxr)
            pltpu.sync_copy(col_hbm.at[wid, pl.ds(p * cpp, cpp)], idxc)
            pltpu.async_copy(hs_hbm.at[idxr.at[0]], rows0, sem0)
            pltpu.async_copy(hs_hbm.at[idxr.at[1]], rows1, sem1)

            def step(k2, carry):
                j = 2 * k2
                pltpu.make_async_copy(hs_hbm.at[idxr.at[j]], rows0,
                                      sem0).wait()
                pltpu.sync_copy(rows0, acc.at[idxc.at[j]], add=True)

                @pl.when(j + 2 < cpp)
                def _():
                    pltpu.async_copy(hs_hbm.at[idxr.at[j + 2]], rows0, sem0)

                pltpu.make_async_copy(hs_hbm.at[idxr.at[j + 1]], rows1,
                                      sem1).wait()
                pltpu.sync_copy(rows1, acc.at[idxc.at[j + 1]], add=True)

                @pl.when(j + 3 < cpp)
                def _():
                    pltpu.async_copy(hs_hbm.at[idxr.at[j + 3]], rows1, sem1)

                return carry

            lax.fori_loop(0, cpp // 2, step, 0)
        plsc.subcore_barrier()
        pltpu.sync_copy(acc.at[pl.ds(base, ROWS_PER_TILE)],
                        out_hbm.at[cid, pl.ds(base, ROWS_PER_TILE)])

    return k


def _build_degree():
    @functools.partial(
        pl.kernel,
        out_type=jax.ShapeDtypeStruct((NC, NPAD, 16), jnp.float32),
        mesh=_sc_mesh(),
        scratch_types=[
            pltpu.VMEM((CHUNKS, 128), jnp.int32),
            pltpu.VMEM((128, 16), jnp.float32),
            pltpu.VMEM_SHARED((NPAD, 16), jnp.float32),
        ],
        compiler_params=pltpu.CompilerParams(use_tc_tiling_on_sc=False),
    )
    def k(col_hbm, zeros_hbm, ones_hbm, out_hbm, idxc, buf, acc):
        cid = lax.axis_index("c")
        sid = lax.axis_index("s")
        wid = sid * NC + cid
        base = sid * ROWS_PER_TILE
        pltpu.sync_copy(zeros_hbm, buf)
        for kk in range(ROWS_PER_TILE // 128):
            pltpu.sync_copy(buf, acc.at[pl.ds(base + kk * 128, 128)])
        pltpu.sync_copy(col_hbm.at[wid], idxc)
        pltpu.sync_copy(ones_hbm, buf)
        plsc.subcore_barrier()

        def step(j, carry):
            pltpu.sync_copy(buf, acc.at[idxc.at[j]], add=True)
            return carry

        lax.fori_loop(0, CHUNKS, step, 0)
        plsc.subcore_barrier()
        for kk in range(ROWS_PER_TILE // 128):
            sl = pl.ds(base + kk * 128, 128)
            pltpu.sync_copy(acc.at[sl], buf)
            pltpu.sync_copy(buf, out_hbm.at[cid, sl])

    return k


def _mm_body(x, w, d0, d1, o, dv):
    dinv = lax.rsqrt(d0[...] + d1[...] + 1.0)
    dv[...] = dinv
    o[...] = jnp.dot(x[...], w[...],
                     preferred_element_type=jnp.float32) * dinv


_mm_call = pl.pallas_call(
    _mm_body,
    grid=(NPAD // BM,),
    in_specs=[
        pl.BlockSpec((BM, D), lambda i: (i, 0)),
        pl.BlockSpec((D, H), lambda i: (0, 0)),
        pl.BlockSpec((BM, 1), lambda i: (i, 0)),
        pl.BlockSpec((BM, 1), lambda i: (i, 0)),
    ],
    out_specs=[pl.BlockSpec((BM, H), lambda i: (i, 0)),
               pl.BlockSpec((BM, 1), lambda i: (i, 0))],
    out_shape=[jax.ShapeDtypeStruct((NPAD, H), jnp.float32),
               jax.ShapeDtypeStruct((NPAD, 1), jnp.float32)],
)


def _build_combine_mm(has_res, hin, hout):
    def body(t0, t1, hs, dinv, b, *rest):
        if has_res:
            res, w, xo, ho = rest
        else:
            w, xo, ho = rest
        u = dinv[...] * (t0[...] + t1[...] + hs[...]) + b[...]
        if has_res:
            u = u + res[...]
        xn = jnp.where(u > 0, u, jnp.exp(jnp.minimum(u, 0.0)) - 1.0)
        xo[...] = xn
        ho[...] = jnp.dot(xn, w[...],
                          preferred_element_type=jnp.float32) * dinv[...]

    blk = pl.BlockSpec((BM, hin), lambda i: (i, 0))
    in_specs = [blk, blk, blk,
                pl.BlockSpec((BM, 1), lambda i: (i, 0)),
                pl.BlockSpec((1, hin), lambda i: (0, 0))]
    if has_res:
        in_specs.append(blk)
    in_specs.append(pl.BlockSpec((hin, hout), lambda i: (0, 0)))
    return pl.pallas_call(
        body,
        grid=(NPAD // BM,),
        in_specs=in_specs,
        out_specs=[blk, pl.BlockSpec((BM, hout), lambda i: (i, 0))],
        out_shape=[jax.ShapeDtypeStruct((NPAD, hin), jnp.float32),
                   jax.ShapeDtypeStruct((NPAD, hout), jnp.float32)],
    )


def _final_body(t0, t1, hs, dinv, b, o):
    o[...] = dinv[...] * (t0[...] + t1[...] + hs[...]) + b[...]


_final_call = pl.pallas_call(
    _final_body,
    grid=(NPAD // BM,),
    in_specs=[
        pl.BlockSpec((BM, C), lambda i: (i, 0)),
        pl.BlockSpec((BM, C), lambda i: (i, 0)),
        pl.BlockSpec((BM, C), lambda i: (i, 0)),
        pl.BlockSpec((BM, 1), lambda i: (i, 0)),
        pl.BlockSpec((1, C), lambda i: (0, 0)),
    ],
    out_specs=pl.BlockSpec((BM, C), lambda i: (i, 0)),
    out_shape=jax.ShapeDtypeStruct((NPAD, C), jnp.float32),
)


def kernel(x, edge_index, batch, W1, b1, W2, b2, W3, b3):
    f32 = jnp.float32
    row = edge_index[0].astype(jnp.int32)
    col = edge_index[1].astype(jnp.int32)
    pad = N + (jnp.arange(EPAD - E, dtype=jnp.int32) % (NPAD - N))
    rowp = jnp.concatenate([row, pad]).reshape(NW, CHUNKS, 128)
    colp = jnp.concatenate([col, pad]).reshape(NW, CHUNKS, 128)
    zeros128 = jnp.zeros((128, 128), f32)
    zeros64 = jnp.zeros((128, 64), f32)
    zeros16 = jnp.zeros((128, 16), f32)
    ones16 = jnp.ones((128, 16), f32)
    x_pad = jnp.pad(x, ((0, NPAD - N), (0, 0)))

    degp = _build_degree()(colp, zeros16, ones16)
    d0 = degp[0, :, :1]
    d1 = degp[1, :, :1]

    hs1, dinv = _mm_call(x_pad, W1, d0, d1)
    t1 = _build_segsum(H)(hs1, rowp, colp, zeros128)
    x1, hs2 = _build_combine_mm(False, H, H)(
        t1[0], t1[1], hs1, dinv, b1.reshape(1, H), W2)
    t2 = _build_segsum(H)(hs2, rowp, colp, zeros128)
    x2, hs3 = _build_combine_mm(True, H, C)(
        t2[0], t2[1], hs2, dinv, b2.reshape(1, H), x1, W3)
    t3 = _build_segsum(C)(hs3, rowp, colp, zeros64)
    out = _final_call(t3[0], t3[1], hs3, dinv, b3.reshape(1, C))
    return out[:N]


# speedup vs baseline: 3.5391x; 1.0552x over previous
import functools

import jax
import jax.numpy as jnp
from jax import lax
from jax.experimental import pallas as pl
from jax.experimental.pallas import tpu as pltpu
from jax.experimental.pallas import tpu_sc as plsc

N = 10000
E = 320000
D = 128
H = 128
C = 64

NC = 2
NS = 16
NW = NC * NS
NPAD = 10240
EPW = 10240
CHUNKS = EPW // 128
EPAD = EPW * NW
ROWS_PER_TILE = NPAD // NS
BM = 1024


def _sc_mesh():
    return plsc.VectorSubcoreMesh(core_axis_name="c", subcore_axis_name="s")


def _build_segsum(width):
    @functools.partial(
        pl.kernel,
        out_type=jax.ShapeDtypeStruct((NC, NPAD, width), jnp.float32),
        mesh=_sc_mesh(),
        scratch_types=[
            pltpu.VMEM((CHUNKS // 2, 128), jnp.int32),
            pltpu.VMEM((CHUNKS // 2, 128), jnp.int32),
            pltpu.VMEM((128, width), jnp.float32),
            pltpu.VMEM((128, width), jnp.float32),
            pltpu.VMEM_SHARED((NPAD, width), jnp.float32),
            pltpu.SemaphoreType.DMA,
            pltpu.SemaphoreType.DMA,
        ],
        compiler_params=pltpu.CompilerParams(use_tc_tiling_on_sc=False),
    )
    def k(hs_hbm, row_hbm, col_hbm, zeros_hbm, out_hbm, idxr, idxc, rows0,
          rows1, acc, sem0, sem1):
        cid = lax.axis_index("c")
        sid = lax.axis_index("s")
        wid = sid * NC + cid
        base = sid * ROWS_PER_TILE
        cpp = CHUNKS // 2

        @pl.when(cid == 0)
        def _():
            pltpu.sync_copy(hs_hbm.at[pl.ds(base, ROWS_PER_TILE)],
                            acc.at[pl.ds(base, ROWS_PER_TILE)])

        @pl.when(cid != 0)
        def _():
            pltpu.sync_copy(zeros_hbm, rows0)
            for kk in range(ROWS_PER_TILE // 128):
                pltpu.sync_copy(rows0, acc.at[pl.ds(base + kk * 128, 128)])

        plsc.subcore_barrier()

        for p in range(2):
            pltpu.sync_copy(row_hbm.at[wid, pl.ds(p * cpp, cpp)], idxr)
            pltpu.sync_copy(col_hbm.at[wid, pl.ds(p * cpp, cpp)], idxc)
            pltpu.async_copy(hs_hbm.at[idxr.at[0]], rows0, sem0)
            pltpu.async_copy(hs_hbm.at[idxr.at[1]], rows1, sem1)

            def step(k2, carry):
                j = 2 * k2
                pltpu.make_async_copy(hs_hbm.at[idxr.at[j]], rows0,
                                      sem0).wait()
                pltpu.sync_copy(rows0, acc.at[idxc.at[j]], add=True)

                @pl.when(j + 2 < cpp)
                def _():
                    pltpu.async_copy(hs_hbm.at[idxr.at[j + 2]], rows0, sem0)

                pltpu.make_async_copy(hs_hbm.at[idxr.at[j + 1]], rows1,
                                      sem1).wait()
                pltpu.sync_copy(rows1, acc.at[idxc.at[j + 1]], add=True)

                @pl.when(j + 3 < cpp)
                def _():
                    pltpu.async_copy(hs_hbm.at[idxr.at[j + 3]], rows1, sem1)

                return carry

            lax.fori_loop(0, cpp // 2, step, 0)
        plsc.subcore_barrier()
        pltpu.sync_copy(acc.at[pl.ds(base, ROWS_PER_TILE)],
                        out_hbm.at[cid, pl.ds(base, ROWS_PER_TILE)])

    return k


def _build_degree():
    @functools.partial(
        pl.kernel,
        out_type=jax.ShapeDtypeStruct((NC, NPAD, 16), jnp.float32),
        mesh=_sc_mesh(),
        scratch_types=[
            pltpu.VMEM((CHUNKS, 128), jnp.int32),
            pltpu.VMEM((128, 16), jnp.float32),
            pltpu.VMEM_SHARED((NPAD, 16), jnp.float32),
        ],
        compiler_params=pltpu.CompilerParams(use_tc_tiling_on_sc=False),
    )
    def k(col_hbm, zeros_hbm, ones_hbm, out_hbm, idxc, buf, acc):
        cid = lax.axis_index("c")
        sid = lax.axis_index("s")
        wid = sid * NC + cid
        base = sid * ROWS_PER_TILE
        pltpu.sync_copy(zeros_hbm, buf)
        for kk in range(ROWS_PER_TILE // 128):
            pltpu.sync_copy(buf, acc.at[pl.ds(base + kk * 128, 128)])
        pltpu.sync_copy(col_hbm.at[wid], idxc)
        pltpu.sync_copy(ones_hbm, buf)
        plsc.subcore_barrier()

        def step(j, carry):
            pltpu.sync_copy(buf, acc.at[idxc.at[j]], add=True)
            return carry

        lax.fori_loop(0, CHUNKS, step, 0)
        plsc.subcore_barrier()
        for kk in range(ROWS_PER_TILE // 128):
            sl = pl.ds(base + kk * 128, 128)
            pltpu.sync_copy(acc.at[sl], buf)
            pltpu.sync_copy(buf, out_hbm.at[cid, sl])

    return k


def _mm_body(x, w, d0, d1, o, dv):
    dinv = lax.rsqrt(d0[...] + d1[...] + 1.0)
    dv[...] = dinv
    o[...] = jnp.dot(x[...], w[...],
                     preferred_element_type=jnp.float32) * dinv


_mm_call = pl.pallas_call(
    _mm_body,
    grid=(NPAD // BM,),
    in_specs=[
        pl.BlockSpec((BM, D), lambda i: (i, 0)),
        pl.BlockSpec((D, H), lambda i: (0, 0)),
        pl.BlockSpec((BM, 1), lambda i: (i, 0)),
        pl.BlockSpec((BM, 1), lambda i: (i, 0)),
    ],
    out_specs=[pl.BlockSpec((BM, H), lambda i: (i, 0)),
               pl.BlockSpec((BM, 1), lambda i: (i, 0))],
    out_shape=[jax.ShapeDtypeStruct((NPAD, H), jnp.float32),
               jax.ShapeDtypeStruct((NPAD, 1), jnp.float32)],
)


def _build_combine_mm(has_res, hin, hout):
    def body(tp, dinv, b, *rest):
        if has_res:
            res, w, xo, ho = rest
        else:
            w, xo, ho = rest
        t = tp[...]
        u = dinv[...] * (t[0] + t[1]) + b[...]
        if has_res:
            u = u + res[...]
        xn = jnp.where(u > 0, u, jnp.exp(jnp.minimum(u, 0.0)) - 1.0)
        xo[...] = xn
        ho[...] = jnp.dot(xn, w[...],
                          preferred_element_type=jnp.float32) * dinv[...]

    blk = pl.BlockSpec((BM, hin), lambda i: (i, 0))
    in_specs = [pl.BlockSpec((2, BM, hin), lambda i: (0, i, 0)),
                pl.BlockSpec((BM, 1), lambda i: (i, 0)),
                pl.BlockSpec((1, hin), lambda i: (0, 0))]
    if has_res:
        in_specs.append(blk)
    in_specs.append(pl.BlockSpec((hin, hout), lambda i: (0, 0)))
    return pl.pallas_call(
        body,
        grid=(NPAD // BM,),
        in_specs=in_specs,
        out_specs=[blk, pl.BlockSpec((BM, hout), lambda i: (i, 0))],
        out_shape=[jax.ShapeDtypeStruct((NPAD, hin), jnp.float32),
                   jax.ShapeDtypeStruct((NPAD, hout), jnp.float32)],
    )


def _final_body(tp, dinv, b, o):
    t = tp[...]
    o[...] = dinv[...] * (t[0] + t[1]) + b[...]


_final_call = pl.pallas_call(
    _final_body,
    grid=(NPAD // BM,),
    in_specs=[
        pl.BlockSpec((2, BM, C), lambda i: (0, i, 0)),
        pl.BlockSpec((BM, 1), lambda i: (i, 0)),
        pl.BlockSpec((1, C), lambda i: (0, 0)),
    ],
    out_specs=pl.BlockSpec((BM, C), lambda i: (i, 0)),
    out_shape=jax.ShapeDtypeStruct((NPAD, C), jnp.float32),
)


def kernel(x, edge_index, batch, W1, b1, W2, b2, W3, b3):
    f32 = jnp.float32
    row = edge_index[0].astype(jnp.int32)
    col = edge_index[1].astype(jnp.int32)
    pad = N + (jnp.arange(EPAD - E, dtype=jnp.int32) % (NPAD - N))
    rowp = jnp.concatenate([row, pad]).reshape(NW, CHUNKS, 128)
    colp = jnp.concatenate([col, pad]).reshape(NW, CHUNKS, 128)
    zeros128 = jnp.zeros((128, 128), f32)
    zeros64 = jnp.zeros((128, 64), f32)
    zeros16 = jnp.zeros((128, 16), f32)
    ones16 = jnp.ones((128, 16), f32)
    x_pad = jnp.pad(x, ((0, NPAD - N), (0, 0)))

    degp = _build_degree()(colp, zeros16, ones16)
    d0 = degp[0, :, :1]
    d1 = degp[1, :, :1]

    hs1, dinv = _mm_call(x_pad, W1, d0, d1)
    t1 = _build_segsum(H)(hs1, rowp, colp, zeros128)
    x1, hs2 = _build_combine_mm(False, H, H)(
        t1, dinv, b1.reshape(1, H), W2)
    t2 = _build_segsum(H)(hs2, rowp, colp, zeros128)
    x2, hs3 = _build_combine_mm(True, H, C)(
        t2, dinv, b2.reshape(1, H), x1, W3)
    t3 = _build_segsum(C)(hs3, rowp, colp, zeros64)
    out = _final_call(t3, dinv, b3.reshape(1, C))
    return out[:N]
